# Initial kernel scaffold; baseline (speedup 1.0000x reference)
import functools

import jax
import jax.numpy as jnp
from jax import lax
from jax.experimental import pallas as pl
from jax.experimental.pallas import tpu as pltpu
from jax.experimental.pallas import tpu_sc as plsc

NC = 2
NS = 16
CH = 128
DEGW = 16


def _sc_edge_sums(x, emb_rel, src2d, et2d, dst2d, npad):
    n, d = x.shape
    nchunks = src2d.shape[0]
    ntiles = NC * NS
    cpt = nchunks // ntiles
    rpt = npad // NS
    zc = rpt // CH
    mesh = plsc.VectorSubcoreMesh(core_axis_name="c", subcore_axis_name="s")

    @functools.partial(
        pl.kernel,
        out_type=[
            jax.ShapeDtypeStruct((NC, npad, d), jnp.float32),
            jax.ShapeDtypeStruct((NC, npad, DEGW), jnp.float32),
        ],
        mesh=mesh,
        scratch_types=[
            pltpu.VMEM((1, CH), jnp.int32),
            pltpu.VMEM((1, CH), jnp.int32),
            pltpu.VMEM((1, CH), jnp.int32),
            pltpu.VMEM((CH, d), jnp.float32),
            pltpu.VMEM((CH, d), jnp.float32),
            pltpu.VMEM((CH, DEGW), jnp.float32),
            pltpu.VMEM((CH, DEGW), jnp.float32),
            pltpu.VMEM_SHARED((npad, d), jnp.float32),
            pltpu.VMEM_SHARED((npad, DEGW), jnp.float32),
        ],
    )
    def sc_kernel(x_hbm, rel_hbm, src_hbm, et_hbm, dst_hbm, s_out, deg_out,
                  src_v, et_v, dst_v, xrows, relrows, ones_v, zdeg, s_sh, deg_sh):
        cid = lax.axis_index("c")
        sid = lax.axis_index("s")
        wid = sid * NC + cid

        zero16 = jnp.zeros((16,), jnp.float32)
        one16 = jnp.ones((16,), jnp.float32)

        @pl.loop(0, CH)
        def _(i):
            for j in range(d // 16):
                xrows[i, pl.ds(j * 16, 16)] = zero16
            ones_v[i, pl.ds(0, DEGW)] = one16
            zdeg[i, pl.ds(0, DEGW)] = zero16

        row0 = sid * rpt
        for j in range(zc):
            pltpu.sync_copy(xrows, s_sh.at[pl.ds(row0 + j * CH, CH)])
            pltpu.sync_copy(zdeg, deg_sh.at[pl.ds(row0 + j * CH, CH)])
        plsc.subcore_barrier()

        @pl.loop(0, cpt)
        def _(k):
            c = wid * cpt + k
            pltpu.sync_copy(src_hbm.at[pl.ds(c, 1)], src_v)
            pltpu.sync_copy(et_hbm.at[pl.ds(c, 1)], et_v)
            pltpu.sync_copy(dst_hbm.at[pl.ds(c, 1)], dst_v)
            pltpu.sync_copy(x_hbm.at[src_v.at[0]], xrows)
            pltpu.sync_copy(rel_hbm.at[et_v.at[0]], relrows)

            @pl.loop(0, CH)
            def _(i):
                for j in range(d // 16):
                    sl = pl.ds(j * 16, 16)
                    xrows[i, sl] = xrows[i, sl] + relrows[i, sl]

            pltpu.sync_copy(xrows, s_sh.at[dst_v.at[0]], add=True)
            pltpu.sync_copy(ones_v, deg_sh.at[dst_v.at[0]], add=True)

        plsc.subcore_barrier()
        pltpu.sync_copy(s_sh.at[pl.ds(row0, rpt)], s_out.at[cid].at[pl.ds(row0, rpt)])
        pltpu.sync_copy(deg_sh.at[pl.ds(row0, rpt)], deg_out.at[cid].at[pl.ds(row0, rpt)])

    return sc_kernel(x, emb_rel, src2d, et2d, dst2d)


def _tc_body(s_ref, deg_ref, x_ref, norm_ref, wn_ref, wl_ref, we_ref, o_ref):
    s = s_ref[0] + s_ref[1]
    agg = jnp.dot(s, wn_ref[...], preferred_element_type=jnp.float32)
    xb = x_ref[...]
    lm_loop = jnp.dot(xb, wl_ref[...], preferred_element_type=jnp.float32)
    lm_ev = jnp.dot(xb, we_ref[...], preferred_element_type=jnp.float32)
    deg = deg_ref[0, :, 0:1] + deg_ref[1, :, 0:1]
    o_ref[...] = agg * norm_ref[...] + jnp.where(deg > 0.0, lm_loop, lm_ev)


def kernel(x, norm, emb_rel, weight_neighbor, loop_weight, evolve_loop_weight,
           src, dst, etype):
    n, d = x.shape
    e = src.shape[0]
    group = CH * NC * NS
    e_pad = ((e + group - 1) // group) * group
    pad = e_pad - e
    if pad:
        src = jnp.concatenate([src, jnp.zeros((pad,), src.dtype)])
        etype = jnp.concatenate([etype, jnp.zeros((pad,), etype.dtype)])
        dst = jnp.concatenate([dst, jnp.full((pad,), n, dst.dtype)])
    src2d = src.reshape(e_pad // CH, CH)
    et2d = etype.reshape(e_pad // CH, CH)
    dst2d = dst.reshape(e_pad // CH, CH)
    npad = ((n + 1 + NS * CH - 1) // (NS * CH)) * (NS * CH)

    s_parts, deg_parts = _sc_edge_sums(x, emb_rel, src2d, et2d, dst2d, npad)

    bt = 2000
    nblocks = n // bt
    return pl.pallas_call(
        _tc_body,
        grid=(nblocks,),
        in_specs=[
            pl.BlockSpec((NC, bt, d), lambda i: (0, i, 0)),
            pl.BlockSpec((NC, bt, DEGW), lambda i: (0, i, 0)),
            pl.BlockSpec((bt, d), lambda i: (i, 0)),
            pl.BlockSpec((bt, 1), lambda i: (i, 0)),
            pl.BlockSpec((d, d), lambda i: (0, 0)),
            pl.BlockSpec((d, d), lambda i: (0, 0)),
            pl.BlockSpec((d, d), lambda i: (0, 0)),
        ],
        out_specs=pl.BlockSpec((bt, d), lambda i: (i, 0)),
        out_shape=jax.ShapeDtypeStruct((n, d), jnp.float32),
    )(s_parts, deg_parts, x, norm, weight_neighbor, loop_weight,
      evolve_loop_weight)


# speedup vs baseline: 3.5636x; 3.5636x over previous
import dataclasses
import functools

import jax
import jax.numpy as jnp
from jax import lax
from jax.experimental import pallas as pl
from jax.experimental.pallas import tpu as pltpu
from jax.experimental.pallas import tpu_sc as plsc

NC = 2
NS = 16
CH = 128
DEGW = 16


def _sc_edge_sums(x, emb_rel, src2d, et2d, dst2d, npad):
    n, d = x.shape
    nchunks = src2d.shape[0]
    ntiles = NC * NS
    cpt = nchunks // ntiles
    rpt = npad // NS
    zc = rpt // CH
    mesh = plsc.VectorSubcoreMesh(core_axis_name="c", subcore_axis_name="s")

    @functools.partial(
        pl.kernel,
        out_type=jax.ShapeDtypeStruct((NC, npad, d), jnp.float32),
        mesh=mesh,
        scratch_types=[
            pltpu.VMEM((1, CH), jnp.int32),
            pltpu.VMEM((1, CH), jnp.int32),
            pltpu.VMEM((1, CH), jnp.int32),
            pltpu.VMEM((CH, d), jnp.float32),
            pltpu.VMEM_SHARED((npad, d), jnp.float32),
        ],
    )
    def sc_rows(x_hbm, rel_hbm, src_hbm, et_hbm, dst_hbm, s_out,
                src_v, et_v, dst_v, rows, s_sh):
        cid = lax.axis_index("c")
        sid = lax.axis_index("s")
        wid = sid * NC + cid
        zero16 = jnp.zeros((16,), jnp.float32)

        @pl.loop(0, CH)
        def _(i):
            for j in range(d // 16):
                rows[i, pl.ds(j * 16, 16)] = zero16

        row0 = sid * rpt
        for j in range(zc):
            pltpu.sync_copy(rows, s_sh.at[pl.ds(row0 + j * CH, CH)])
        rem = rpt - zc * CH
        if rem:
            pltpu.sync_copy(rows.at[pl.ds(0, rem)],
                            s_sh.at[pl.ds(row0 + zc * CH, rem)])
        plsc.subcore_barrier()

        @pl.loop(0, cpt)
        def _(k):
            c = wid * cpt + k
            pltpu.sync_copy(src_hbm.at[pl.ds(c, 1)], src_v)
            pltpu.sync_copy(et_hbm.at[pl.ds(c, 1)], et_v)
            pltpu.sync_copy(dst_hbm.at[pl.ds(c, 1)], dst_v)
            pltpu.sync_copy(x_hbm.at[src_v.at[0]], rows)
            pltpu.sync_copy(rows, s_sh.at[dst_v.at[0]], add=True)
            pltpu.sync_copy(rel_hbm.at[et_v.at[0]], rows)
            pltpu.sync_copy(rows, s_sh.at[dst_v.at[0]], add=True)

        plsc.subcore_barrier()
        pltpu.sync_copy(s_sh.at[pl.ds(row0, rpt)], s_out.at[cid].at[pl.ds(row0, rpt)])

    cp = pltpu.CompilerParams()
    if "needs_layout_passes" in pltpu.CompilerParams.__dataclass_fields__:
        cp = dataclasses.replace(cp, needs_layout_passes=False)

    @functools.partial(
        pl.kernel,
        out_type=jax.ShapeDtypeStruct((ntiles, npad), jnp.float32),
        mesh=mesh,
        compiler_params=cp,
        scratch_types=[
            pltpu.VMEM((1, CH), jnp.int32),
            pltpu.VMEM((npad,), jnp.float32),
        ],
    )
    def sc_deg(dst_hbm, deg_out, dst_v, deg_local):
        cid = lax.axis_index("c")
        sid = lax.axis_index("s")
        wid = sid * NC + cid
        zero16 = jnp.zeros((16,), jnp.float32)
        one16 = jnp.ones((16,), jnp.float32)

        @pl.loop(0, npad // 16)
        def _(i):
            deg_local[pl.ds(i * 16, 16)] = zero16

        @pl.loop(0, cpt)
        def _(k):
            c = wid * cpt + k
            pltpu.sync_copy(dst_hbm.at[pl.ds(c, 1)], dst_v)
            for j in range(CH // 16):
                idx16 = dst_v[0, pl.ds(j * 16, 16)]
                plsc.addupdate_scatter(deg_local, [idx16], one16)

        pltpu.sync_copy(deg_local, deg_out.at[wid])

    return sc_rows(x, emb_rel, src2d, et2d, dst2d), sc_deg(dst2d)


def _tc_body(s_ref, deg_ref, x_ref, norm_ref, wn_ref, wl_ref, we_ref, o_ref):
    s = s_ref[0] + s_ref[1]
    agg = jnp.dot(s, wn_ref[...], preferred_element_type=jnp.float32)
    xb = x_ref[...]
    lm_loop = jnp.dot(xb, wl_ref[...], preferred_element_type=jnp.float32)
    lm_ev = jnp.dot(xb, we_ref[...], preferred_element_type=jnp.float32)
    deg = jnp.sum(deg_ref[...], axis=1, keepdims=True)
    o_ref[...] = agg * norm_ref[...] + jnp.where(deg > 0.0, lm_loop, lm_ev)


def kernel(x, norm, emb_rel, weight_neighbor, loop_weight, evolve_loop_weight,
           src, dst, etype):
    n, d = x.shape
    e = src.shape[0]
    group = CH * NC * NS
    e_pad = ((e + group - 1) // group) * group
    pad = e_pad - e
    if pad:
        src = jnp.concatenate([src, jnp.zeros((pad,), src.dtype)])
        etype = jnp.concatenate([etype, jnp.zeros((pad,), etype.dtype)])
        dst = jnp.concatenate([dst, jnp.full((pad,), n, dst.dtype)])
    src2d = src.reshape(e_pad // CH, CH)
    et2d = etype.reshape(e_pad // CH, CH)
    dst2d = dst.reshape(e_pad // CH, CH)
    npad = ((n + 1 + NS * 8 - 1) // (NS * 8)) * (NS * 8)

    s_parts, deg_parts = _sc_edge_sums(x, emb_rel, src2d, et2d, dst2d, npad)
    deg_t = deg_parts.T

    bt = 2000
    nblocks = n // bt
    return pl.pallas_call(
        _tc_body,
        grid=(nblocks,),
        in_specs=[
            pl.BlockSpec((NC, bt, d), lambda i: (0, i, 0)),
            pl.BlockSpec((bt, NC * NS), lambda i: (i, 0)),
            pl.BlockSpec((bt, d), lambda i: (i, 0)),
            pl.BlockSpec((bt, 1), lambda i: (i, 0)),
            pl.BlockSpec((d, d), lambda i: (0, 0)),
            pl.BlockSpec((d, d), lambda i: (0, 0)),
            pl.BlockSpec((d, d), lambda i: (0, 0)),
        ],
        out_specs=pl.BlockSpec((bt, d), lambda i: (i, 0)),
        out_shape=jax.ShapeDtypeStruct((n, d), jnp.float32),
    )(s_parts, deg_t, x, norm, weight_neighbor, loop_weight,
      evolve_loop_weight)
